# Initial kernel scaffold; baseline (speedup 1.0000x reference)
import jax
import jax.numpy as jnp
from jax.experimental import pallas as pl


def kernel(inputs, embedding):
    raise NotImplementedError("write your pallas kernel here")


# speedup vs baseline: 1.1018x; 1.1018x over previous
import jax
import jax.numpy as jnp
from jax import lax
from jax.experimental import pallas as pl

N_EMB = 8192
DIM = 128
R = 256


def _emit_body(idx_ref, eb_ref, enc_ref, q_ref):
    idx = idx_ref[0, 0, :]
    cols = lax.broadcasted_iota(jnp.int32, (R, N_EMB), 1)
    enc = (cols == idx[:, None]).astype(jnp.float32)
    enc_ref[...] = enc
    q = lax.dot_general(enc.astype(jnp.bfloat16), eb_ref[...],
                        (((1,), (0,)), ((), ())),
                        preferred_element_type=jnp.float32)
    q_ref[...] = q.T[None]


def kernel(inputs, embedding):
    b, h, w, c = inputs.shape
    flat = inputs.reshape(-1, c)
    n = flat.shape[0]
    steps = n // R
    rows_per_batch = h * w
    blocks_per_batch = rows_per_batch // R

    distances = (jnp.sum(flat ** 2, axis=1, keepdims=True)
                 + jnp.sum(embedding ** 2, axis=1)
                 - 2.0 * jnp.matmul(flat, embedding.T))
    idx = jnp.argmin(distances, axis=1).astype(jnp.int32).reshape(steps, 1, R)

    eb = embedding.astype(jnp.bfloat16)

    enc, qout = pl.pallas_call(
        _emit_body,
        grid=(steps,),
        in_specs=[
            pl.BlockSpec((1, 1, R), lambda i: (i, 0, 0)),
            pl.BlockSpec((N_EMB, DIM), lambda i: (0, 0)),
        ],
        out_specs=[
            pl.BlockSpec((R, N_EMB), lambda i: (i, 0)),
            pl.BlockSpec(
                (1, DIM, R),
                lambda i: (i // blocks_per_batch, 0, i % blocks_per_batch),
            ),
        ],
        out_shape=[
            jax.ShapeDtypeStruct((n, N_EMB), jnp.float32),
            jax.ShapeDtypeStruct((b, DIM, rows_per_batch), jnp.float32),
        ],
    )(idx, eb)
    return (qout.reshape(b, DIM, h, w), enc)
